# baseline (speedup 1.0000x reference)
import functools

import jax
import jax.numpy as jnp
from jax import lax
from jax.experimental import pallas as pl
from jax.experimental.pallas import tpu as pltpu
from jax.experimental.pallas import tpu_sc as plsc

NUM_EXPERTS = 64
TOP_K = 2
D_MODEL = 768
D_HID = 3072
N_TOK = 4096
BLK = 128
NB = 128
P = NB * BLK
N_PAIRS = N_TOK * TOP_K


def _gate_body(x_ref, wgt_ref, bg_ref, e0_ref, e1_ref, w0_ref, w1_ref,
               xb_ref):
    xb = x_ref[...].astype(jnp.bfloat16)
    xb_ref[...] = xb
    logits = jnp.dot(xb, wgt_ref[...].astype(jnp.bfloat16),
                     preferred_element_type=jnp.float32)
    logits = logits + bg_ref[...]
    m0 = jnp.max(logits, axis=1, keepdims=True)
    a0 = jnp.argmax(logits, axis=1)[:, None]
    lane = lax.broadcasted_iota(jnp.int32, logits.shape, 1)
    masked = jnp.where(lane == a0, -1e30, logits)
    m1 = jnp.max(masked, axis=1, keepdims=True)
    a1 = jnp.argmax(masked, axis=1)[:, None]
    z = jnp.exp(m1 - m0)
    w0 = 1.0 / (1.0 + z)
    e0_ref[...] = a0.astype(jnp.int32)
    e1_ref[...] = a1.astype(jnp.int32)
    w0_ref[...] = w0
    w1_ref[...] = 1.0 - w0


def _gate(x_flat, wg_t, bg_row):
    return pl.pallas_call(
        _gate_body,
        out_shape=[
            jax.ShapeDtypeStruct((N_TOK, 1), jnp.int32),
            jax.ShapeDtypeStruct((N_TOK, 1), jnp.int32),
            jax.ShapeDtypeStruct((N_TOK, 1), jnp.float32),
            jax.ShapeDtypeStruct((N_TOK, 1), jnp.float32),
            jax.ShapeDtypeStruct((N_TOK, D_MODEL), jnp.bfloat16),
        ],
    )(x_flat, wg_t, bg_row)


def _route(e0, e1, w0, w1):
    ef = jnp.concatenate([e0, e1], axis=1).reshape(-1)
    wf = jnp.concatenate([w0, w1], axis=1).reshape(-1)
    iota = jnp.arange(N_PAIRS, dtype=jnp.int32)
    key = ef * N_PAIRS + iota
    skey = jnp.sort(key)
    sorted_e = skey // N_PAIRS
    order = skey - sorted_e * N_PAIRS
    counts = jnp.bincount(ef, length=NUM_EXPERTS)
    padded = ((counts + BLK - 1) // BLK) * BLK
    ends = jnp.cumsum(padded)
    starts = ends - padded
    cum_counts = jnp.cumsum(counts) - counts
    rank = iota - cum_counts[sorted_e].astype(jnp.int32)
    pos_sorted = starts[sorted_e].astype(jnp.int32) + rank
    total = ends[-1]
    na = (total // BLK).astype(jnp.int32).reshape(1)
    src_tok = jnp.zeros((P,), jnp.int32).at[pos_sorted].set(order // TOP_K)
    pos_flat = jnp.zeros((N_PAIRS,), jnp.int32).at[order].set(pos_sorted)
    w_sorted = jnp.zeros((P,), jnp.float32).at[pos_flat].set(wf)
    pos_mat = pos_flat.reshape(N_TOK, TOP_K)
    pos01 = jnp.concatenate([pos_mat[:, 0], pos_mat[:, 1]])
    blk_base = jnp.arange(NB, dtype=jnp.int32) * BLK
    block_expert = jnp.searchsorted(
        ends, jnp.minimum(blk_base, total - 1), side="right").astype(jnp.int32)
    return src_tok, w_sorted.reshape(P, 1), pos01, block_expert, na


D_HALF = D_MODEL // 2


def _bf16_to_i32(a):
    return lax.bitcast_convert_type(
        a.reshape(a.shape[0], a.shape[1] // 2, 2), jnp.int32)


def _i32_to_bf16(a):
    return lax.bitcast_convert_type(a, jnp.bfloat16).reshape(
        a.shape[0], a.shape[1] * 2)


def _sc_gather(table, idx, n_rows, chunk):
    mesh = plsc.VectorSubcoreMesh(core_axis_name="c", subcore_axis_name="s")
    n_tiles = 32
    per_tile = n_rows // n_tiles
    n_chunks = per_tile // chunk

    @functools.partial(
        pl.kernel,
        mesh=mesh,
        out_type=jax.ShapeDtypeStruct((n_rows, D_HALF), jnp.int32),
        scratch_types=[
            pltpu.VMEM((per_tile,), jnp.int32),
            pltpu.VMEM((chunk, D_HALF), jnp.int32),
            pltpu.VMEM((chunk, D_HALF), jnp.int32),
            pltpu.SemaphoreType.DMA,
            pltpu.SemaphoreType.DMA,
        ],
    )
    def k(table_hbm, idx_hbm, out_hbm, idx_v, rows_a, rows_b, sem_a, sem_b):
        wid = lax.axis_index("s") * 2 + lax.axis_index("c")
        base = wid * per_tile
        pltpu.sync_copy(idx_hbm.at[pl.ds(base, per_tile)], idx_v)
        bufs = (rows_a, rows_b)
        sems = (sem_a, sem_b)

        def gcopy(c):
            return pltpu.make_async_copy(
                table_hbm.at[idx_v.at[pl.ds(c * chunk, chunk)]],
                bufs[c % 2], sems[c % 2])

        gcopy(0).start()
        for c in range(n_chunks):
            if c + 1 < n_chunks:
                gcopy(c + 1).start()
            gcopy(c).wait()
            pltpu.sync_copy(bufs[c % 2],
                            out_hbm.at[pl.ds(base + c * chunk, chunk)])

    return k(table, idx)


def _ffn_body(be_ref, na_ref, xs_ref, ws_ref, w1_ref, b1_ref, w2_ref, b2_ref,
              y_ref):
    b = pl.program_id(0)

    @pl.when(b < na_ref[0])
    def _():
        h = jnp.dot(xs_ref[...].astype(jnp.float32), w1_ref[0],
                    preferred_element_type=jnp.float32)
        h = h + b1_ref[0]
        h = 0.5 * h * (1.0 + lax.erf(h * 0.7071067811865476))
        y = jnp.dot(h, w2_ref[0], preferred_element_type=jnp.float32)
        y = y + b2_ref[0]
        y_ref[...] = (y * ws_ref[...]).astype(jnp.bfloat16)


def _ffn(block_expert, na, xs, w_sorted, W1, b1, W2, b2):
    grid_spec = pltpu.PrefetchScalarGridSpec(
        num_scalar_prefetch=2,
        grid=(NB,),
        in_specs=[
            pl.BlockSpec((BLK, D_MODEL), lambda b, be, na: (b, 0)),
            pl.BlockSpec((BLK, 1), lambda b, be, na: (b, 0)),
            pl.BlockSpec((1, D_MODEL, D_HID), lambda b, be, na: (be[b], 0, 0)),
            pl.BlockSpec((1, 1, D_HID), lambda b, be, na: (be[b], 0, 0)),
            pl.BlockSpec((1, D_HID, D_MODEL), lambda b, be, na: (be[b], 0, 0)),
            pl.BlockSpec((1, 1, D_MODEL), lambda b, be, na: (be[b], 0, 0)),
        ],
        out_specs=pl.BlockSpec((BLK, D_MODEL), lambda b, be, na: (b, 0)),
    )
    return pl.pallas_call(
        _ffn_body,
        grid_spec=grid_spec,
        out_shape=jax.ShapeDtypeStruct((P, D_MODEL), jnp.bfloat16),
        compiler_params=pltpu.CompilerParams(
            dimension_semantics=("arbitrary",),
            vmem_limit_bytes=100 * 1024 * 1024,
        ),
    )(block_expert, na, xs, w_sorted,
      W1, b1.reshape(NUM_EXPERTS, 1, D_HID),
      W2, b2.reshape(NUM_EXPERTS, 1, D_MODEL))


def _add_body(a_ref, b_ref, o_ref):
    o_ref[...] = (a_ref[...].astype(jnp.float32)
                  + b_ref[...].astype(jnp.float32))


def _combine(yg):
    n_blk = 8
    rows = N_TOK // n_blk
    return pl.pallas_call(
        _add_body,
        grid=(n_blk,),
        in_specs=[
            pl.BlockSpec((rows, D_MODEL), lambda i: (i, 0)),
            pl.BlockSpec((rows, D_MODEL), lambda i: (i + n_blk, 0)),
        ],
        out_specs=pl.BlockSpec((rows, D_MODEL), lambda i: (i, 0)),
        out_shape=jax.ShapeDtypeStruct((N_TOK, D_MODEL), jnp.float32),
    )(yg, yg)


def kernel(x, Wg, bg, W1, b1, W2, b2):
    bsz, seq_len, dim = x.shape
    x_flat = x.reshape(N_TOK, D_MODEL)
    e0, e1, w0, w1, xb = _gate(x_flat, Wg.T, bg.reshape(1, NUM_EXPERTS))
    src_tok, w_sorted, pos01, block_expert, na = _route(e0, e1, w0, w1)
    xs = _i32_to_bf16(_sc_gather(_bf16_to_i32(xb), src_tok, P, 128))
    y = _ffn(block_expert, na, xs, w_sorted, W1, b1, W2, b2)
    yg = _i32_to_bf16(_sc_gather(_bf16_to_i32(y), pos01, N_PAIRS, 128))
    out = _combine(yg)
    return out.reshape(bsz, seq_len, dim)


# speedup vs baseline: 1.3371x; 1.3371x over previous
import functools

import jax
import jax.numpy as jnp
from jax import lax
from jax.experimental import pallas as pl
from jax.experimental.pallas import tpu as pltpu
from jax.experimental.pallas import tpu_sc as plsc

NUM_EXPERTS = 64
TOP_K = 2
D_MODEL = 768
D_HID = 3072
N_TOK = 4096
BLK = 128
NB = 128
P = NB * BLK
N_PAIRS = N_TOK * TOP_K


def _gate_body(x_ref, wgt_ref, bg_ref, e0_ref, e1_ref, w0_ref, w1_ref):
    logits = jnp.dot(x_ref[...].astype(jnp.bfloat16),
                     wgt_ref[...].astype(jnp.bfloat16),
                     preferred_element_type=jnp.float32)
    logits = logits + bg_ref[...]
    m0 = jnp.max(logits, axis=1, keepdims=True)
    a0 = jnp.argmax(logits, axis=1)[:, None]
    lane = lax.broadcasted_iota(jnp.int32, logits.shape, 1)
    masked = jnp.where(lane == a0, -1e30, logits)
    m1 = jnp.max(masked, axis=1, keepdims=True)
    a1 = jnp.argmax(masked, axis=1)[:, None]
    z = jnp.exp(m1 - m0)
    w0 = 1.0 / (1.0 + z)
    e0_ref[...] = a0.astype(jnp.int32)
    e1_ref[...] = a1.astype(jnp.int32)
    w0_ref[...] = w0
    w1_ref[...] = 1.0 - w0


def _gate(x_flat, wg_t, bg_row):
    return pl.pallas_call(
        _gate_body,
        out_shape=[
            jax.ShapeDtypeStruct((N_TOK, 1), jnp.int32),
            jax.ShapeDtypeStruct((N_TOK, 1), jnp.int32),
            jax.ShapeDtypeStruct((N_TOK, 1), jnp.float32),
            jax.ShapeDtypeStruct((N_TOK, 1), jnp.float32),
        ],
    )(x_flat, wg_t, bg_row)


def _route(e0, e1, w0, w1):
    ef = jnp.concatenate([e0, e1], axis=1).reshape(-1)
    wf = jnp.concatenate([w0, w1], axis=1).reshape(-1)
    iota = jnp.arange(N_PAIRS, dtype=jnp.int32)
    key = ef * N_PAIRS + iota
    skey = jnp.sort(key)
    sorted_e = skey // N_PAIRS
    order = skey - sorted_e * N_PAIRS
    counts = jnp.bincount(ef, length=NUM_EXPERTS)
    padded = ((counts + BLK - 1) // BLK) * BLK
    ends = jnp.cumsum(padded)
    starts = ends - padded
    cum_counts = jnp.cumsum(counts) - counts
    rank = iota - cum_counts[sorted_e].astype(jnp.int32)
    pos_sorted = starts[sorted_e].astype(jnp.int32) + rank
    total = ends[-1]
    na = (total // BLK).astype(jnp.int32).reshape(1)
    src_tok = jnp.zeros((P,), jnp.int32).at[pos_sorted].set(order // TOP_K)
    pos_flat = jnp.zeros((N_PAIRS,), jnp.int32).at[order].set(pos_sorted)
    w_sorted = jnp.zeros((P,), jnp.float32).at[pos_flat].set(wf)
    pos_mat = pos_flat.reshape(N_TOK, TOP_K)
    pos01 = jnp.concatenate([pos_mat[:, 0], pos_mat[:, 1]])
    blk_base = jnp.arange(NB, dtype=jnp.int32) * BLK
    block_expert = jnp.searchsorted(
        ends, jnp.minimum(blk_base, total - 1), side="right").astype(jnp.int32)
    return src_tok, w_sorted.reshape(P, 1), pos01, block_expert, na


def _sc_gather(table, idx, n_rows, chunk):
    mesh = plsc.VectorSubcoreMesh(core_axis_name="c", subcore_axis_name="s")
    n_tiles = 32
    per_tile = n_rows // n_tiles
    n_chunks = per_tile // chunk

    @functools.partial(
        pl.kernel,
        mesh=mesh,
        out_type=jax.ShapeDtypeStruct((n_rows, D_MODEL), jnp.float32),
        scratch_types=[
            pltpu.VMEM((per_tile,), jnp.int32),
            pltpu.VMEM((chunk, D_MODEL), jnp.float32),
            pltpu.VMEM((chunk, D_MODEL), jnp.float32),
            pltpu.SemaphoreType.DMA,
            pltpu.SemaphoreType.DMA,
        ],
        compiler_params=pltpu.CompilerParams(use_tc_tiling_on_sc=False),
    )
    def k(table_hbm, idx_hbm, out_hbm, idx_v, rows_a, rows_b, sem_a, sem_b):
        wid = lax.axis_index("s") * 2 + lax.axis_index("c")
        base = wid * per_tile
        pltpu.sync_copy(idx_hbm.at[pl.ds(base, per_tile)], idx_v)
        bufs = (rows_a, rows_b)
        sems = (sem_a, sem_b)

        def gcopy(c):
            return pltpu.make_async_copy(
                table_hbm.at[idx_v.at[pl.ds(c * chunk, chunk)]],
                bufs[c % 2], sems[c % 2])

        gcopy(0).start()
        for c in range(n_chunks):
            if c + 1 < n_chunks:
                gcopy(c + 1).start()
            gcopy(c).wait()
            pltpu.sync_copy(bufs[c % 2],
                            out_hbm.at[pl.ds(base + c * chunk, chunk)])

    return k(table, idx)


def _ffn_body(be_ref, na_ref, xs_ref, ws_ref, w1_ref, b1_ref, w2_ref, b2_ref,
              y_ref):
    b = pl.program_id(0)

    @pl.when(b < na_ref[0])
    def _():
        h = jnp.dot(xs_ref[...], w1_ref[0],
                    preferred_element_type=jnp.float32)
        h = h + b1_ref[0]
        h = 0.5 * h * (1.0 + lax.erf(h * 0.7071067811865476))
        y = jnp.dot(h, w2_ref[0], preferred_element_type=jnp.float32)
        y = y + b2_ref[0]
        y_ref[...] = y * ws_ref[...]


def _ffn(block_expert, na, xs, w_sorted, W1, b1, W2, b2):
    grid_spec = pltpu.PrefetchScalarGridSpec(
        num_scalar_prefetch=2,
        grid=(NB,),
        in_specs=[
            pl.BlockSpec((BLK, D_MODEL), lambda b, be, na: (b, 0)),
            pl.BlockSpec((BLK, 1), lambda b, be, na: (b, 0)),
            pl.BlockSpec((1, D_MODEL, D_HID), lambda b, be, na: (be[b], 0, 0)),
            pl.BlockSpec((1, 1, D_HID), lambda b, be, na: (be[b], 0, 0)),
            pl.BlockSpec((1, D_HID, D_MODEL), lambda b, be, na: (be[b], 0, 0)),
            pl.BlockSpec((1, 1, D_MODEL), lambda b, be, na: (be[b], 0, 0)),
        ],
        out_specs=pl.BlockSpec((BLK, D_MODEL), lambda b, be, na: (b, 0)),
    )
    return pl.pallas_call(
        _ffn_body,
        grid_spec=grid_spec,
        out_shape=jax.ShapeDtypeStruct((P, D_MODEL), jnp.float32),
        compiler_params=pltpu.CompilerParams(
            dimension_semantics=("arbitrary",),
            vmem_limit_bytes=100 * 1024 * 1024,
        ),
    )(block_expert, na, xs, w_sorted,
      W1, b1.reshape(NUM_EXPERTS, 1, D_HID),
      W2, b2.reshape(NUM_EXPERTS, 1, D_MODEL))


def _add_body(a_ref, b_ref, o_ref):
    o_ref[...] = a_ref[...] + b_ref[...]


def _combine(yg):
    n_blk = 8
    rows = N_TOK // n_blk
    return pl.pallas_call(
        _add_body,
        grid=(n_blk,),
        in_specs=[
            pl.BlockSpec((rows, D_MODEL), lambda i: (i, 0)),
            pl.BlockSpec((rows, D_MODEL), lambda i: (i + n_blk, 0)),
        ],
        out_specs=pl.BlockSpec((rows, D_MODEL), lambda i: (i, 0)),
        out_shape=jax.ShapeDtypeStruct((N_TOK, D_MODEL), jnp.float32),
    )(yg, yg)


def kernel(x, Wg, bg, W1, b1, W2, b2):
    bsz, seq_len, dim = x.shape
    x_flat = x.reshape(N_TOK, D_MODEL)
    e0, e1, w0, w1 = _gate(x_flat, Wg.T, bg.reshape(1, NUM_EXPERTS))
    src_tok, w_sorted, pos01, block_expert, na = _route(e0, e1, w0, w1)
    xs = _sc_gather(x_flat, src_tok, P, 64)
    y = _ffn(block_expert, na, xs, w_sorted, W1, b1, W2, b2)
    yg = _sc_gather(y, pos01, N_PAIRS, 64)
    out = _combine(yg)
    return out.reshape(bsz, seq_len, dim)


# speedup vs baseline: 1.3515x; 1.0108x over previous
import functools

import jax
import jax.numpy as jnp
from jax import lax
from jax.experimental import pallas as pl
from jax.experimental.pallas import tpu as pltpu
from jax.experimental.pallas import tpu_sc as plsc

NUM_EXPERTS = 64
TOP_K = 2
D_MODEL = 768
D_HID = 3072
N_TOK = 4096
BLK = 128
NB = 128
P = NB * BLK
N_PAIRS = N_TOK * TOP_K


def _gate_body(x_ref, wgt_ref, bg_ref, e0_ref, e1_ref, w0_ref, w1_ref,
               xc_ref):
    xc_ref[...] = x_ref[...]
    logits = jnp.dot(x_ref[...].astype(jnp.bfloat16),
                     wgt_ref[...].astype(jnp.bfloat16),
                     preferred_element_type=jnp.float32)
    logits = logits + bg_ref[...]
    m0 = jnp.max(logits, axis=1, keepdims=True)
    a0 = jnp.argmax(logits, axis=1)[:, None]
    lane = lax.broadcasted_iota(jnp.int32, logits.shape, 1)
    masked = jnp.where(lane == a0, -1e30, logits)
    m1 = jnp.max(masked, axis=1, keepdims=True)
    a1 = jnp.argmax(masked, axis=1)[:, None]
    z = jnp.exp(m1 - m0)
    w0 = 1.0 / (1.0 + z)
    e0_ref[...] = a0.astype(jnp.int32)
    e1_ref[...] = a1.astype(jnp.int32)
    w0_ref[...] = w0
    w1_ref[...] = 1.0 - w0


def _gate(x_flat, wg_t, bg_row):
    return pl.pallas_call(
        _gate_body,
        out_shape=[
            jax.ShapeDtypeStruct((N_TOK, 1), jnp.int32),
            jax.ShapeDtypeStruct((N_TOK, 1), jnp.int32),
            jax.ShapeDtypeStruct((N_TOK, 1), jnp.float32),
            jax.ShapeDtypeStruct((N_TOK, 1), jnp.float32),
            jax.ShapeDtypeStruct((N_TOK, D_MODEL), jnp.float32),
        ],
    )(x_flat, wg_t, bg_row)


def _route(e0, e1, w0, w1):
    ef = jnp.concatenate([e0, e1], axis=1).reshape(-1)
    wf = jnp.concatenate([w0, w1], axis=1).reshape(-1)
    iota = jnp.arange(N_PAIRS, dtype=jnp.int32)
    key = ef * N_PAIRS + iota
    skey = jnp.sort(key)
    sorted_e = skey // N_PAIRS
    order = skey - sorted_e * N_PAIRS
    counts = jnp.bincount(ef, length=NUM_EXPERTS)
    padded = ((counts + BLK - 1) // BLK) * BLK
    ends = jnp.cumsum(padded)
    starts = ends - padded
    cum_counts = jnp.cumsum(counts) - counts
    rank = iota - cum_counts[sorted_e].astype(jnp.int32)
    pos_sorted = starts[sorted_e].astype(jnp.int32) + rank
    total = ends[-1]
    na = (total // BLK).astype(jnp.int32).reshape(1)
    src_tok = jnp.zeros((P,), jnp.int32).at[pos_sorted].set(order // TOP_K)
    pos_flat = jnp.zeros((N_PAIRS,), jnp.int32).at[order].set(pos_sorted)
    w_sorted = jnp.zeros((P,), jnp.float32).at[pos_flat].set(wf)
    pos_mat = pos_flat.reshape(N_TOK, TOP_K)
    pos01 = jnp.concatenate([pos_mat[:, 0], pos_mat[:, 1]])
    blk_base = jnp.arange(NB, dtype=jnp.int32) * BLK
    block_expert = jnp.searchsorted(
        ends, jnp.minimum(blk_base, total - 1), side="right").astype(jnp.int32)
    return src_tok, w_sorted.reshape(P, 1), pos01, block_expert, na


def _sc_gather(table, idx, n_rows, chunk):
    mesh = plsc.VectorSubcoreMesh(core_axis_name="c", subcore_axis_name="s")
    n_tiles = 32
    per_tile = n_rows // n_tiles
    n_chunks = per_tile // chunk

    @functools.partial(
        pl.kernel,
        mesh=mesh,
        out_type=jax.ShapeDtypeStruct((n_rows, D_MODEL), jnp.float32),
        scratch_types=[
            pltpu.VMEM((per_tile,), jnp.int32),
            pltpu.VMEM((chunk, D_MODEL), jnp.float32),
            pltpu.VMEM((chunk, D_MODEL), jnp.float32),
            pltpu.SemaphoreType.DMA,
            pltpu.SemaphoreType.DMA,
        ],
        compiler_params=pltpu.CompilerParams(use_tc_tiling_on_sc=False),
    )
    def k(table_hbm, idx_hbm, out_hbm, idx_v, rows_a, rows_b, sem_a, sem_b):
        wid = lax.axis_index("s") * 2 + lax.axis_index("c")
        base = wid * per_tile
        pltpu.sync_copy(idx_hbm.at[pl.ds(base, per_tile)], idx_v)
        bufs = (rows_a, rows_b)
        sems = (sem_a, sem_b)

        def gcopy(c):
            return pltpu.make_async_copy(
                table_hbm.at[idx_v.at[pl.ds(c * chunk, chunk)]],
                bufs[c % 2], sems[c % 2])

        gcopy(0).start()
        for c in range(n_chunks):
            if c + 1 < n_chunks:
                gcopy(c + 1).start()
            gcopy(c).wait()
            pltpu.sync_copy(bufs[c % 2],
                            out_hbm.at[pl.ds(base + c * chunk, chunk)])

    return k(table, idx)


def _ffn_body(be_ref, na_ref, xs_ref, ws_ref, w1_ref, b1_ref, w2_ref, b2_ref,
              y_ref):
    b = pl.program_id(0)

    @pl.when(b < na_ref[0])
    def _():
        h = jnp.dot(xs_ref[...], w1_ref[0],
                    preferred_element_type=jnp.float32)
        h = h + b1_ref[0]
        h = 0.5 * h * (1.0 + lax.erf(h * 0.7071067811865476))
        y = jnp.dot(h, w2_ref[0], preferred_element_type=jnp.float32)
        y = y + b2_ref[0]
        y_ref[...] = y * ws_ref[...]


def _ffn(block_expert, na, xs, w_sorted, W1, b1, W2, b2):
    grid_spec = pltpu.PrefetchScalarGridSpec(
        num_scalar_prefetch=2,
        grid=(NB,),
        in_specs=[
            pl.BlockSpec((BLK, D_MODEL), lambda b, be, na: (b, 0)),
            pl.BlockSpec((BLK, 1), lambda b, be, na: (b, 0)),
            pl.BlockSpec((1, D_MODEL, D_HID), lambda b, be, na: (be[b], 0, 0)),
            pl.BlockSpec((1, 1, D_HID), lambda b, be, na: (be[b], 0, 0)),
            pl.BlockSpec((1, D_HID, D_MODEL), lambda b, be, na: (be[b], 0, 0)),
            pl.BlockSpec((1, 1, D_MODEL), lambda b, be, na: (be[b], 0, 0)),
        ],
        out_specs=pl.BlockSpec((BLK, D_MODEL), lambda b, be, na: (b, 0)),
    )
    return pl.pallas_call(
        _ffn_body,
        grid_spec=grid_spec,
        out_shape=jax.ShapeDtypeStruct((P, D_MODEL), jnp.float32),
        compiler_params=pltpu.CompilerParams(
            dimension_semantics=("arbitrary",),
            vmem_limit_bytes=100 * 1024 * 1024,
        ),
    )(block_expert, na, xs, w_sorted,
      W1, b1.reshape(NUM_EXPERTS, 1, D_HID),
      W2, b2.reshape(NUM_EXPERTS, 1, D_MODEL))


def _add_body(a_ref, b_ref, o_ref):
    o_ref[...] = a_ref[...] + b_ref[...]


def _combine(yg):
    n_blk = 8
    rows = N_TOK // n_blk
    return pl.pallas_call(
        _add_body,
        grid=(n_blk,),
        in_specs=[
            pl.BlockSpec((rows, D_MODEL), lambda i: (i, 0)),
            pl.BlockSpec((rows, D_MODEL), lambda i: (i + n_blk, 0)),
        ],
        out_specs=pl.BlockSpec((rows, D_MODEL), lambda i: (i, 0)),
        out_shape=jax.ShapeDtypeStruct((N_TOK, D_MODEL), jnp.float32),
    )(yg, yg)


def kernel(x, Wg, bg, W1, b1, W2, b2):
    bsz, seq_len, dim = x.shape
    x_flat = x.reshape(N_TOK, D_MODEL)
    e0, e1, w0, w1, x_copy = _gate(x_flat, Wg.T, bg.reshape(1, NUM_EXPERTS))
    src_tok, w_sorted, pos01, block_expert, na = _route(e0, e1, w0, w1)
    xs = _sc_gather(x_copy, src_tok, P, 64)
    y = _ffn(block_expert, na, xs, w_sorted, W1, b1, W2, b2)
    yg = _sc_gather(y, pos01, N_PAIRS, 64)
    out = _combine(yg)
    return out.reshape(bsz, seq_len, dim)


# speedup vs baseline: 1.7877x; 1.3227x over previous
import functools

import jax
import jax.numpy as jnp
from jax import lax
from jax.experimental import pallas as pl
from jax.experimental.pallas import tpu as pltpu
from jax.experimental.pallas import tpu_sc as plsc

NUM_EXPERTS = 64
TOP_K = 2
D_MODEL = 768
D_HID = 3072
N_TOK = 4096
BLK = 128
NB = 128
P = NB * BLK
N_PAIRS = N_TOK * TOP_K


def _gate_body(x_ref, wgt_ref, bg_ref, e0_ref, e1_ref, w0_ref, w1_ref,
               xc_ref):
    xc_ref[...] = x_ref[...]
    logits = jnp.dot(x_ref[...].astype(jnp.bfloat16),
                     wgt_ref[...].astype(jnp.bfloat16),
                     preferred_element_type=jnp.float32)
    logits = logits + bg_ref[...]
    m0 = jnp.max(logits, axis=1, keepdims=True)
    a0 = jnp.argmax(logits, axis=1)[:, None]
    lane = lax.broadcasted_iota(jnp.int32, logits.shape, 1)
    masked = jnp.where(lane == a0, -1e30, logits)
    m1 = jnp.max(masked, axis=1, keepdims=True)
    a1 = jnp.argmax(masked, axis=1)[:, None]
    z = jnp.exp(m1 - m0)
    w0 = 1.0 / (1.0 + z)
    e0_ref[...] = a0.astype(jnp.int32)
    e1_ref[...] = a1.astype(jnp.int32)
    w0_ref[...] = w0
    w1_ref[...] = 1.0 - w0


def _gate(x_flat, wg_t, bg_row):
    return pl.pallas_call(
        _gate_body,
        out_shape=[
            jax.ShapeDtypeStruct((N_TOK, 1), jnp.int32),
            jax.ShapeDtypeStruct((N_TOK, 1), jnp.int32),
            jax.ShapeDtypeStruct((N_TOK, 1), jnp.float32),
            jax.ShapeDtypeStruct((N_TOK, 1), jnp.float32),
            jax.ShapeDtypeStruct((N_TOK, D_MODEL), jnp.float32),
        ],
    )(x_flat, wg_t, bg_row)


def _route(e0, e1, w0, w1):
    ef = jnp.concatenate([e0, e1], axis=1).reshape(-1)
    wf = jnp.concatenate([w0, w1], axis=1).reshape(-1)
    iota = jnp.arange(N_PAIRS, dtype=jnp.int32)
    key = ef * N_PAIRS + iota
    skey = jnp.sort(key)
    sorted_e = skey // N_PAIRS
    order = skey - sorted_e * N_PAIRS
    counts = jnp.bincount(ef, length=NUM_EXPERTS)
    padded = ((counts + BLK - 1) // BLK) * BLK
    ends = jnp.cumsum(padded)
    starts = ends - padded
    cum_counts = jnp.cumsum(counts) - counts
    rank = iota - cum_counts[sorted_e].astype(jnp.int32)
    pos_sorted = starts[sorted_e].astype(jnp.int32) + rank
    total = ends[-1]
    na = (total // BLK).astype(jnp.int32).reshape(1)
    pad_src = jnp.arange(P, dtype=jnp.int32) % N_TOK
    src_tok = pad_src.at[pos_sorted].set(order // TOP_K)
    pos_flat = jnp.zeros((N_PAIRS,), jnp.int32).at[order].set(pos_sorted)
    w_sorted = jnp.zeros((P,), jnp.float32).at[pos_flat].set(wf)
    pos_mat = pos_flat.reshape(N_TOK, TOP_K)
    pos01 = jnp.concatenate([pos_mat[:, 0], pos_mat[:, 1]])
    blk_base = jnp.arange(NB, dtype=jnp.int32) * BLK
    block_expert = jnp.searchsorted(
        ends, jnp.minimum(blk_base, total - 1), side="right").astype(jnp.int32)
    return src_tok, w_sorted.reshape(P, 1), pos01, block_expert, na


def _sc_gather(table, idx, n_rows, chunk):
    mesh = plsc.VectorSubcoreMesh(core_axis_name="c", subcore_axis_name="s")
    n_tiles = 32
    per_tile = n_rows // n_tiles
    n_chunks = per_tile // chunk

    @functools.partial(
        pl.kernel,
        mesh=mesh,
        out_type=jax.ShapeDtypeStruct((n_rows, D_MODEL), jnp.float32),
        scratch_types=[
            pltpu.VMEM((per_tile,), jnp.int32),
            pltpu.VMEM((chunk, D_MODEL), jnp.float32),
            pltpu.VMEM((chunk, D_MODEL), jnp.float32),
            pltpu.SemaphoreType.DMA,
            pltpu.SemaphoreType.DMA,
        ],
        compiler_params=pltpu.CompilerParams(use_tc_tiling_on_sc=False),
    )
    def k(table_hbm, idx_hbm, out_hbm, idx_v, rows_a, rows_b, sem_a, sem_b):
        wid = lax.axis_index("s") * 2 + lax.axis_index("c")
        base = wid * per_tile
        pltpu.sync_copy(idx_hbm.at[pl.ds(base, per_tile)], idx_v)
        bufs = (rows_a, rows_b)
        sems = (sem_a, sem_b)

        def gcopy(c):
            return pltpu.make_async_copy(
                table_hbm.at[idx_v.at[pl.ds(c * chunk, chunk)]],
                bufs[c % 2], sems[c % 2])

        gcopy(0).start()
        for c in range(n_chunks):
            if c + 1 < n_chunks:
                gcopy(c + 1).start()
            gcopy(c).wait()
            pltpu.sync_copy(bufs[c % 2],
                            out_hbm.at[pl.ds(base + c * chunk, chunk)])

    return k(table, idx)


def _ffn_body(be_ref, na_ref, xs_ref, ws_ref, w1_ref, b1_ref, w2_ref, b2_ref,
              y_ref):
    b = pl.program_id(0)

    @pl.when(b < na_ref[0])
    def _():
        h = jnp.dot(xs_ref[...], w1_ref[0],
                    preferred_element_type=jnp.float32)
        h = h + b1_ref[0]
        h = 0.5 * h * (1.0 + lax.erf(h * 0.7071067811865476))
        y = jnp.dot(h, w2_ref[0], preferred_element_type=jnp.float32)
        y = y + b2_ref[0]
        y_ref[...] = y * ws_ref[...]


def _ffn(block_expert, na, xs, w_sorted, W1, b1, W2, b2):
    grid_spec = pltpu.PrefetchScalarGridSpec(
        num_scalar_prefetch=2,
        grid=(NB,),
        in_specs=[
            pl.BlockSpec((BLK, D_MODEL), lambda b, be, na: (b, 0)),
            pl.BlockSpec((BLK, 1), lambda b, be, na: (b, 0)),
            pl.BlockSpec((1, D_MODEL, D_HID), lambda b, be, na: (be[b], 0, 0)),
            pl.BlockSpec((1, 1, D_HID), lambda b, be, na: (be[b], 0, 0)),
            pl.BlockSpec((1, D_HID, D_MODEL), lambda b, be, na: (be[b], 0, 0)),
            pl.BlockSpec((1, 1, D_MODEL), lambda b, be, na: (be[b], 0, 0)),
        ],
        out_specs=pl.BlockSpec((BLK, D_MODEL), lambda b, be, na: (b, 0)),
    )
    return pl.pallas_call(
        _ffn_body,
        grid_spec=grid_spec,
        out_shape=jax.ShapeDtypeStruct((P, D_MODEL), jnp.float32),
        compiler_params=pltpu.CompilerParams(
            dimension_semantics=("arbitrary",),
            vmem_limit_bytes=100 * 1024 * 1024,
        ),
    )(block_expert, na, xs, w_sorted,
      W1, b1.reshape(NUM_EXPERTS, 1, D_HID),
      W2, b2.reshape(NUM_EXPERTS, 1, D_MODEL))


def _add_body(a_ref, b_ref, o_ref):
    o_ref[...] = a_ref[...] + b_ref[...]


def _combine(yg):
    n_blk = 8
    rows = N_TOK // n_blk
    return pl.pallas_call(
        _add_body,
        grid=(n_blk,),
        in_specs=[
            pl.BlockSpec((rows, D_MODEL), lambda i: (i, 0)),
            pl.BlockSpec((rows, D_MODEL), lambda i: (i + n_blk, 0)),
        ],
        out_specs=pl.BlockSpec((rows, D_MODEL), lambda i: (i, 0)),
        out_shape=jax.ShapeDtypeStruct((N_TOK, D_MODEL), jnp.float32),
    )(yg, yg)


def kernel(x, Wg, bg, W1, b1, W2, b2):
    bsz, seq_len, dim = x.shape
    x_flat = x.reshape(N_TOK, D_MODEL)
    e0, e1, w0, w1, x_copy = _gate(x_flat, Wg.T, bg.reshape(1, NUM_EXPERTS))
    src_tok, w_sorted, pos01, block_expert, na = _route(e0, e1, w0, w1)
    xs = _sc_gather(x_copy, src_tok, P, 64)
    y = _ffn(block_expert, na, xs, w_sorted, W1, b1, W2, b2)
    yg = _sc_gather(y, pos01, N_PAIRS, 64)
    out = _combine(yg)
    return out.reshape(bsz, seq_len, dim)


# speedup vs baseline: 1.8222x; 1.0193x over previous
import functools

import jax
import jax.numpy as jnp
from jax import lax
from jax.experimental import pallas as pl
from jax.experimental.pallas import tpu as pltpu
from jax.experimental.pallas import tpu_sc as plsc

NUM_EXPERTS = 64
TOP_K = 2
D_MODEL = 768
D_HID = 3072
N_TOK = 4096
BLK = 256
NB = 96
P = NB * BLK
N_PAIRS = N_TOK * TOP_K


def _gate_body(x_ref, wgt_ref, bg_ref, e0_ref, e1_ref, w0_ref, w1_ref,
               xc_ref):
    xc_ref[...] = x_ref[...]
    logits = jnp.dot(x_ref[...].astype(jnp.bfloat16),
                     wgt_ref[...].astype(jnp.bfloat16),
                     preferred_element_type=jnp.float32)
    logits = logits + bg_ref[...]
    m0 = jnp.max(logits, axis=1, keepdims=True)
    a0 = jnp.argmax(logits, axis=1)[:, None]
    lane = lax.broadcasted_iota(jnp.int32, logits.shape, 1)
    masked = jnp.where(lane == a0, -1e30, logits)
    m1 = jnp.max(masked, axis=1, keepdims=True)
    a1 = jnp.argmax(masked, axis=1)[:, None]
    z = jnp.exp(m1 - m0)
    w0 = 1.0 / (1.0 + z)
    e0_ref[...] = a0.astype(jnp.int32)
    e1_ref[...] = a1.astype(jnp.int32)
    w0_ref[...] = w0
    w1_ref[...] = 1.0 - w0


def _gate(x_flat, wg_t, bg_row):
    return pl.pallas_call(
        _gate_body,
        out_shape=[
            jax.ShapeDtypeStruct((N_TOK, 1), jnp.int32),
            jax.ShapeDtypeStruct((N_TOK, 1), jnp.int32),
            jax.ShapeDtypeStruct((N_TOK, 1), jnp.float32),
            jax.ShapeDtypeStruct((N_TOK, 1), jnp.float32),
            jax.ShapeDtypeStruct((N_TOK, D_MODEL), jnp.float32),
        ],
    )(x_flat, wg_t, bg_row)


def _route(e0, e1, w0, w1):
    ef = jnp.concatenate([e0, e1], axis=1).reshape(-1)
    wf = jnp.concatenate([w0, w1], axis=1).reshape(-1)
    iota = jnp.arange(N_PAIRS, dtype=jnp.int32)
    key = ef * N_PAIRS + iota
    skey = jnp.sort(key)
    sorted_e = skey // N_PAIRS
    order = skey - sorted_e * N_PAIRS
    counts = jnp.bincount(ef, length=NUM_EXPERTS)
    padded = ((counts + BLK - 1) // BLK) * BLK
    ends = jnp.cumsum(padded)
    starts = ends - padded
    cum_counts = jnp.cumsum(counts) - counts
    rank = iota - cum_counts[sorted_e].astype(jnp.int32)
    pos_sorted = starts[sorted_e].astype(jnp.int32) + rank
    total = ends[-1]
    na = (total // BLK).astype(jnp.int32).reshape(1)
    pad_src = jnp.arange(P, dtype=jnp.int32) % N_TOK
    src_tok = pad_src.at[pos_sorted].set(order // TOP_K)
    pos_flat = jnp.zeros((N_PAIRS,), jnp.int32).at[order].set(pos_sorted)
    w_sorted = jnp.zeros((P,), jnp.float32).at[pos_flat].set(wf)
    pos_mat = pos_flat.reshape(N_TOK, TOP_K)
    pos01 = jnp.concatenate([pos_mat[:, 0], pos_mat[:, 1]])
    blk_base = jnp.arange(NB, dtype=jnp.int32) * BLK
    block_expert = jnp.searchsorted(
        ends, jnp.minimum(blk_base, total - 1), side="right").astype(jnp.int32)
    return src_tok, w_sorted.reshape(P, 1), pos01, block_expert, na


def _sc_gather(table, idx, n_rows, chunk):
    mesh = plsc.VectorSubcoreMesh(core_axis_name="c", subcore_axis_name="s")
    n_tiles = 32
    per_tile = n_rows // n_tiles
    n_chunks = per_tile // chunk

    @functools.partial(
        pl.kernel,
        mesh=mesh,
        out_type=jax.ShapeDtypeStruct((n_rows, D_MODEL), jnp.float32),
        scratch_types=[
            pltpu.VMEM((per_tile,), jnp.int32),
            pltpu.VMEM((chunk, D_MODEL), jnp.float32),
            pltpu.VMEM((chunk, D_MODEL), jnp.float32),
            pltpu.SemaphoreType.DMA,
            pltpu.SemaphoreType.DMA,
        ],
        compiler_params=pltpu.CompilerParams(use_tc_tiling_on_sc=False),
    )
    def k(table_hbm, idx_hbm, out_hbm, idx_v, rows_a, rows_b, sem_a, sem_b):
        wid = lax.axis_index("s") * 2 + lax.axis_index("c")
        base = wid * per_tile
        pltpu.sync_copy(idx_hbm.at[pl.ds(base, per_tile)], idx_v)
        bufs = (rows_a, rows_b)
        sems = (sem_a, sem_b)

        def gcopy(c):
            return pltpu.make_async_copy(
                table_hbm.at[idx_v.at[pl.ds(c * chunk, chunk)]],
                bufs[c % 2], sems[c % 2])

        gcopy(0).start()
        for c in range(n_chunks):
            if c + 1 < n_chunks:
                gcopy(c + 1).start()
            gcopy(c).wait()
            pltpu.sync_copy(bufs[c % 2],
                            out_hbm.at[pl.ds(base + c * chunk, chunk)])

    return k(table, idx)


def _ffn_body(be_ref, na_ref, xs_ref, ws_ref, w1_ref, b1_ref, w2_ref, b2_ref,
              y_ref):
    b = pl.program_id(0)

    @pl.when(b < na_ref[0])
    def _():
        h = jnp.dot(xs_ref[...], w1_ref[0],
                    preferred_element_type=jnp.float32)
        h = h + b1_ref[0]
        h = 0.5 * h * (1.0 + lax.erf(h * 0.7071067811865476))
        y = jnp.dot(h, w2_ref[0], preferred_element_type=jnp.float32)
        y = y + b2_ref[0]
        y_ref[...] = y * ws_ref[...]


def _ffn(block_expert, na, xs, w_sorted, W1, b1, W2, b2):
    grid_spec = pltpu.PrefetchScalarGridSpec(
        num_scalar_prefetch=2,
        grid=(NB,),
        in_specs=[
            pl.BlockSpec((BLK, D_MODEL), lambda b, be, na: (b, 0)),
            pl.BlockSpec((BLK, 1), lambda b, be, na: (b, 0)),
            pl.BlockSpec((1, D_MODEL, D_HID), lambda b, be, na: (be[b], 0, 0)),
            pl.BlockSpec((1, 1, D_HID), lambda b, be, na: (be[b], 0, 0)),
            pl.BlockSpec((1, D_HID, D_MODEL), lambda b, be, na: (be[b], 0, 0)),
            pl.BlockSpec((1, 1, D_MODEL), lambda b, be, na: (be[b], 0, 0)),
        ],
        out_specs=pl.BlockSpec((BLK, D_MODEL), lambda b, be, na: (b, 0)),
    )
    return pl.pallas_call(
        _ffn_body,
        grid_spec=grid_spec,
        out_shape=jax.ShapeDtypeStruct((P, D_MODEL), jnp.float32),
        compiler_params=pltpu.CompilerParams(
            dimension_semantics=("arbitrary",),
            vmem_limit_bytes=100 * 1024 * 1024,
        ),
    )(block_expert, na, xs, w_sorted,
      W1, b1.reshape(NUM_EXPERTS, 1, D_HID),
      W2, b2.reshape(NUM_EXPERTS, 1, D_MODEL))


def _add_body(a_ref, b_ref, o_ref):
    o_ref[...] = a_ref[...] + b_ref[...]


def _combine(yg):
    n_blk = 8
    rows = N_TOK // n_blk
    return pl.pallas_call(
        _add_body,
        grid=(n_blk,),
        in_specs=[
            pl.BlockSpec((rows, D_MODEL), lambda i: (i, 0)),
            pl.BlockSpec((rows, D_MODEL), lambda i: (i + n_blk, 0)),
        ],
        out_specs=pl.BlockSpec((rows, D_MODEL), lambda i: (i, 0)),
        out_shape=jax.ShapeDtypeStruct((N_TOK, D_MODEL), jnp.float32),
    )(yg, yg)


def kernel(x, Wg, bg, W1, b1, W2, b2):
    bsz, seq_len, dim = x.shape
    x_flat = x.reshape(N_TOK, D_MODEL)
    e0, e1, w0, w1, x_copy = _gate(x_flat, Wg.T, bg.reshape(1, NUM_EXPERTS))
    src_tok, w_sorted, pos01, block_expert, na = _route(e0, e1, w0, w1)
    xs = _sc_gather(x_copy, src_tok, P, 64)
    y = _ffn(block_expert, na, xs, w_sorted, W1, b1, W2, b2)
    yg = _sc_gather(y, pos01, N_PAIRS, 64)
    out = _combine(yg)
    return out.reshape(bsz, seq_len, dim)
